# Initial kernel scaffold; baseline (speedup 1.0000x reference)
import jax
import jax.numpy as jnp
from jax.experimental import pallas as pl


def kernel(x, edge_index, W_enc, b_enc, W1, b1, W2, b2, Ws, bs, Wt, bt):
    raise NotImplementedError("write your pallas kernel here")


# speedup vs baseline: 6.3802x; 6.3802x over previous
import functools

import jax
import jax.numpy as jnp
from jax import lax
from jax.experimental import pallas as pl
from jax.experimental.pallas import tpu as pltpu
from jax.experimental.pallas import tpu_sc as plsc

N = 10000
E = 320000
D = 128

NC = 2
NS = 16
NW = NC * NS
EPW = E // NW
EB = 80
NB = EPW // EB
RPT = N // NS
L = 16

_mesh = plsc.VectorSubcoreMesh(core_axis_name="c", subcore_axis_name="s")


def _wid(c, s):
    return c * NS + s


def _sc_segsum(h, src, dst):
    @functools.partial(
        pl.kernel,
        out_type=jax.ShapeDtypeStruct((NC, NS, RPT, D), jnp.float32),
        mesh=_mesh,
        compiler_params=pltpu.CompilerParams(needs_layout_passes=False),
        scratch_types=[
            pltpu.VMEM((EB,), jnp.int32),
            pltpu.VMEM((EB,), jnp.int32),
            pltpu.VMEM((EB, D), jnp.float32),
            pltpu.VMEM((125, D), jnp.float32),
            pltpu.VMEM_SHARED((N, D), jnp.float32),
            pltpu.SemaphoreType.DMA,
        ],
    )
    def k(h_hbm, src_hbm, dst_hbm, out_hbm, sidx, didx, rows, zbuf, acc, sem):
        c = lax.axis_index("c")
        s = lax.axis_index("s")
        base = _wid(c, s) * EPW

        def zb(i, _):
            def zb2(j, _):
                zbuf[i, pl.ds(j * L, L)] = jnp.zeros((L,), jnp.float32)
                return 0
            return lax.fori_loop(0, D // L, zb2, 0)
        lax.fori_loop(0, 125, zb, 0)

        def zacc(i, _):
            pltpu.sync_copy(zbuf, acc.at[pl.ds(s * RPT + i * 125, 125)])
            return 0
        lax.fori_loop(0, RPT // 125, zacc, 0)
        plsc.subcore_barrier()

        def body(i, _):
            off = base + i * EB
            pltpu.sync_copy(src_hbm.at[pl.ds(off, EB)], sidx)
            pltpu.sync_copy(dst_hbm.at[pl.ds(off, EB)], didx)
            pltpu.async_copy(h_hbm.at[sidx], rows, sem).wait()
            pltpu.sync_copy(rows, acc.at[didx], add=True)
            return 0
        lax.fori_loop(0, NB, body, 0)
        plsc.subcore_barrier()

        pltpu.sync_copy(acc.at[pl.ds(s * RPT, RPT)], out_hbm.at[c, s])

    return k(h, src, dst).reshape(NC, N, D)


def _sc_dots(zs, zt, src, dst):
    @functools.partial(
        pl.kernel,
        out_type=(jax.ShapeDtypeStruct((E,), jnp.float32),
                  jax.ShapeDtypeStruct((NC, NS, N), jnp.float32)),
        mesh=_mesh,
        compiler_params=pltpu.CompilerParams(needs_layout_passes=False),
        scratch_types=[
            pltpu.VMEM((EB,), jnp.int32),
            pltpu.VMEM((EB,), jnp.int32),
            pltpu.VMEM((EB, D), jnp.float32),
            pltpu.VMEM((EB, D), jnp.float32),
            pltpu.VMEM((EB,), jnp.float32),
            pltpu.VMEM((N,), jnp.float32),
            pltpu.VMEM((L * L,), jnp.float32),
            pltpu.SemaphoreType.DMA,
            pltpu.SemaphoreType.DMA,
        ],
    )
    def k(zs_hbm, zt_hbm, src_hbm, dst_hbm, e_hbm, mxp_hbm,
          sidx, didx, srows, trows, ebuf, mxacc, ps, sem1, sem2):
        c = lax.axis_index("c")
        s = lax.axis_index("s")
        base = _wid(c, s) * EPW
        iota = jnp.arange(L, dtype=jnp.int32)

        def zmx(i, _):
            mxacc[pl.ds(i * L, L)] = jnp.full((L,), -jnp.inf, jnp.float32)
            return 0
        lax.fori_loop(0, N // L, zmx, 0)

        def body(i, _):
            off = base + i * EB
            pltpu.sync_copy(src_hbm.at[pl.ds(off, EB)], sidx)
            pltpu.sync_copy(dst_hbm.at[pl.ds(off, EB)], didx)
            cp1 = pltpu.async_copy(zs_hbm.at[sidx], srows, sem1)
            cp2 = pltpu.async_copy(zt_hbm.at[didx], trows, sem2)
            cp1.wait()
            cp2.wait()
            for g in range(EB // L):
                for j in range(L):
                    acc = jnp.zeros((L,), jnp.float32)
                    for fc in range(D // L):
                        vs = srows[g * L + j, pl.ds(fc * L, L)]
                        vt = trows[g * L + j, pl.ds(fc * L, L)]
                        acc = acc + vs * vt
                    ps[pl.ds(j * L, L)] = acc
                ev = jnp.zeros((L,), jnp.float32)
                for l in range(L):
                    ev = ev + plsc.load_gather(ps, [iota * L + l])
                ebuf[pl.ds(g * L, L)] = ev

                sg = sidx[pl.ds(g * L, L)]
                m = ev
                for r in range(1, L):
                    perm = (iota + r) & (L - 1)
                    sr = sg.at[perm].get(mode="promise_in_bounds")
                    er = ev.at[perm].get(mode="promise_in_bounds")
                    m = jnp.where(sr == sg, jnp.maximum(m, er), m)
                cur = plsc.load_gather(mxacc, [sg])
                plsc.store_scatter(mxacc, [sg], jnp.maximum(cur, m))
            pltpu.sync_copy(ebuf, e_hbm.at[pl.ds(off, EB)])
            return 0
        lax.fori_loop(0, NB, body, 0)

        pltpu.sync_copy(mxacc, mxp_hbm.at[c, s])

    return k(zs, zt, src, dst)


def _sc_expsum(e, src, mx):
    @functools.partial(
        pl.kernel,
        out_type=(jax.ShapeDtypeStruct((E,), jnp.float32),
                  jax.ShapeDtypeStruct((NC, NS, N), jnp.float32)),
        mesh=_mesh,
        compiler_params=pltpu.CompilerParams(needs_layout_passes=False),
        scratch_types=[
            pltpu.VMEM((EB,), jnp.int32),
            pltpu.VMEM((EB,), jnp.float32),
            pltpu.VMEM((EB,), jnp.float32),
            pltpu.VMEM((N,), jnp.float32),
            pltpu.VMEM((N,), jnp.float32),
        ],
    )
    def k(e_hbm, src_hbm, mx_hbm, esh_hbm, dp_hbm,
          sidx, ebuf, oebuf, mxl, dacc):
        c = lax.axis_index("c")
        s = lax.axis_index("s")
        base = _wid(c, s) * EPW

        pltpu.sync_copy(mx_hbm, mxl)

        def zd(i, _):
            dacc[pl.ds(i * L, L)] = jnp.zeros((L,), jnp.float32)
            return 0
        lax.fori_loop(0, N // L, zd, 0)

        def body(i, _):
            off = base + i * EB
            pltpu.sync_copy(src_hbm.at[pl.ds(off, EB)], sidx)
            pltpu.sync_copy(e_hbm.at[pl.ds(off, EB)], ebuf)
            for g in range(EB // L):
                sg = sidx[pl.ds(g * L, L)]
                ev = ebuf[pl.ds(g * L, L)]
                mval = plsc.load_gather(mxl, [sg])
                esh = ev - mval
                oebuf[pl.ds(g * L, L)] = esh
                plsc.addupdate_scatter(dacc, [sg], jnp.exp(esh))
            pltpu.sync_copy(oebuf, esh_hbm.at[pl.ds(off, EB)])
            return 0
        lax.fori_loop(0, NB, body, 0)

        pltpu.sync_copy(dacc, dp_hbm.at[c, s])

    return k(e, src, mx)


def _sc_final(esh, src, ld):
    @functools.partial(
        pl.kernel,
        out_type=jax.ShapeDtypeStruct((E,), jnp.float32),
        mesh=_mesh,
        compiler_params=pltpu.CompilerParams(needs_layout_passes=False),
        scratch_types=[
            pltpu.VMEM((EB,), jnp.int32),
            pltpu.VMEM((EB,), jnp.float32),
            pltpu.VMEM((EB,), jnp.float32),
            pltpu.VMEM((N,), jnp.float32),
        ],
    )
    def k(esh_hbm, src_hbm, ld_hbm, out_hbm, sidx, ebuf, obuf, ldl):
        c = lax.axis_index("c")
        s = lax.axis_index("s")
        base = _wid(c, s) * EPW

        pltpu.sync_copy(ld_hbm, ldl)

        def body(i, _):
            off = base + i * EB
            pltpu.sync_copy(src_hbm.at[pl.ds(off, EB)], sidx)
            pltpu.sync_copy(esh_hbm.at[pl.ds(off, EB)], ebuf)
            for g in range(EB // L):
                sg = sidx[pl.ds(g * L, L)]
                ev = ebuf[pl.ds(g * L, L)]
                obuf[pl.ds(g * L, L)] = ev - plsc.load_gather(ldl, [sg])
            pltpu.sync_copy(obuf, out_hbm.at[pl.ds(off, EB)])
            return 0
        lax.fori_loop(0, NB, body, 0)

    return k(esh, src, ld)


_RB = 400


def _enc(x, W, b):
    def k(x_ref, w_ref, b_ref, o_ref):
        o_ref[...] = jnp.dot(x_ref[...], w_ref[...],
                             preferred_element_type=jnp.float32) + b_ref[...]

    return pl.pallas_call(
        k,
        grid=(N // _RB,),
        in_specs=[
            pl.BlockSpec((_RB, D), lambda i: (i, 0)),
            pl.BlockSpec((D, D), lambda i: (0, 0)),
            pl.BlockSpec((1, D), lambda i: (0, 0)),
        ],
        out_specs=pl.BlockSpec((_RB, D), lambda i: (i, 0)),
        out_shape=jax.ShapeDtypeStruct((N, D), jnp.float32),
    )(x, W, b.reshape(1, D))


def _proc(h, aggp, W1, b1, W2, b2, Ws, bs, Wt, bt):
    def k(h_ref, a_ref, w1, b1r, w2, b2r, ws, bsr, wt, btr, zs_ref, zt_ref):
        z = h_ref[...] + a_ref[0] + a_ref[1]
        z = jnp.dot(z, w1[...], preferred_element_type=jnp.float32) + b1r[...]
        z = jnp.maximum(z, 0.0)
        h2 = jnp.dot(z, w2[...], preferred_element_type=jnp.float32) + b2r[...]
        zs_ref[...] = jnp.dot(h2, ws[...],
                              preferred_element_type=jnp.float32) + bsr[...]
        zt_ref[...] = jnp.dot(h2, wt[...],
                              preferred_element_type=jnp.float32) + btr[...]

    wspec = pl.BlockSpec((D, D), lambda i: (0, 0))
    bspec = pl.BlockSpec((1, D), lambda i: (0, 0))
    return pl.pallas_call(
        k,
        grid=(N // _RB,),
        in_specs=[
            pl.BlockSpec((_RB, D), lambda i: (i, 0)),
            pl.BlockSpec((NC, _RB, D), lambda i: (0, i, 0)),
            wspec, bspec, wspec, bspec, wspec, bspec, wspec, bspec,
        ],
        out_specs=[pl.BlockSpec((_RB, D), lambda i: (i, 0)),
                   pl.BlockSpec((_RB, D), lambda i: (i, 0))],
        out_shape=[jax.ShapeDtypeStruct((N, D), jnp.float32),
                   jax.ShapeDtypeStruct((N, D), jnp.float32)],
    )(h, aggp, W1, b1.reshape(1, D), W2, b2.reshape(1, D),
      Ws, bs.reshape(1, D), Wt, bt.reshape(1, D))


def _combine_max(mxp):
    def k(p_ref, o_ref):
        m = jnp.max(p_ref[...], axis=(0, 1))
        o_ref[...] = jnp.where(jnp.isfinite(m), m, 0.0).reshape(1, N)

    return pl.pallas_call(
        k,
        out_shape=jax.ShapeDtypeStruct((1, N), jnp.float32),
    )(mxp)


def _combine_logsum(dp):
    def k(p_ref, o_ref):
        d = jnp.sum(p_ref[...], axis=(0, 1))
        o_ref[...] = jnp.log(d + 1e-12).reshape(1, N)

    return pl.pallas_call(
        k,
        out_shape=jax.ShapeDtypeStruct((1, N), jnp.float32),
    )(dp)


def kernel(x, edge_index, W_enc, b_enc, W1, b1, W2, b2, Ws, bs, Wt, bt):
    src = edge_index[0]
    dst = edge_index[1]
    h = _enc(x, W_enc, b_enc)
    aggp = _sc_segsum(h, src, dst)
    zs, zt = _proc(h, aggp, W1, b1, W2, b2, Ws, bs, Wt, bt)
    e, mxp = _sc_dots(zs, zt, src, dst)
    mx = _combine_max(mxp).reshape(N)
    esh, dp = _sc_expsum(e, src, mx)
    ld = _combine_logsum(dp).reshape(N)
    return _sc_final(esh, src, ld)


# speedup vs baseline: 12.0679x; 1.8915x over previous
import functools

import jax
import jax.numpy as jnp
from jax import lax
from jax.experimental import pallas as pl
from jax.experimental.pallas import tpu as pltpu
from jax.experimental.pallas import tpu_sc as plsc

N = 10000
E = 320000
D = 128

NC = 2
NS = 16
NW = NC * NS
EPW = E // NW
EB = 80
NB = EPW // EB
RPT = N // NS
L = 16
GPB = EB // L

_mesh = plsc.VectorSubcoreMesh(core_axis_name="c", subcore_axis_name="s")
_sc_params = pltpu.CompilerParams(needs_layout_passes=False)


def _wid(c, s):
    return c * NS + s


def _sc_segsum(h, src3, dst3):
    @functools.partial(
        pl.kernel,
        out_type=jax.ShapeDtypeStruct((NC, NS, RPT, D), jnp.float32),
        mesh=_mesh,
        compiler_params=_sc_params,
        scratch_types=[
            pltpu.VMEM((NB, EB), jnp.int32),
            pltpu.VMEM((1, EPW), jnp.int32),
            pltpu.VMEM((EB, D), jnp.float32),
            pltpu.VMEM((EB, D), jnp.float32),
            pltpu.VMEM((5, D), jnp.float32),
            pltpu.VMEM_SHARED((N, D), jnp.float32),
            pltpu.SemaphoreType.DMA,
            pltpu.SemaphoreType.DMA,
            pltpu.SemaphoreType.DMA,
            pltpu.SemaphoreType.DMA,
        ],
    )
    def k(h_hbm, src_hbm, dst_hbm, out_hbm,
          sidx, didx, rows0, rows1, zbuf, acc, sg0, sg1, sc0, sc1):
        c = lax.axis_index("c")
        s = lax.axis_index("s")
        w = _wid(c, s)

        pltpu.sync_copy(src_hbm.at[w], sidx)
        pltpu.sync_copy(dst_hbm.at[w], didx)

        def zb(i, _):
            def zb2(j, _):
                zbuf[i, pl.ds(j * L, L)] = jnp.zeros((L,), jnp.float32)
                return 0
            return lax.fori_loop(0, D // L, zb2, 0)
        lax.fori_loop(0, 5, zb, 0)

        def zacc(i, _):
            pltpu.sync_copy(zbuf, acc.at[pl.ds(s * RPT + i * 5, 5)])
            return 0
        lax.fori_loop(0, RPT // 5, zacc, 0)
        plsc.subcore_barrier()

        def g_issue(i, buf, sem):
            pltpu.async_copy(h_hbm.at[sidx.at[i]], buf, sem)

        def g_wait(i, buf, sem):
            pltpu.make_async_copy(h_hbm.at[sidx.at[i]], buf, sem).wait()

        def scat(i, buf, sem):
            for g in range(GPB):
                dg = didx[0, pl.ds(i * EB + g * L, L)]
                pltpu.async_copy(buf.at[pl.ds(g * L, L)], acc.at[dg], sem,
                                 add=True)

        def scat_drain(i, buf, sem):
            for g in range(GPB):
                dg = didx[0, pl.ds(i * EB + g * L, L)]
                pltpu.make_async_copy(buf.at[pl.ds(g * L, L)], acc.at[dg],
                                      sem).wait()

        g_issue(0, rows0, sg0)

        def body(kk, _):
            i0 = 2 * kk
            g_issue(i0 + 1, rows1, sg1)
            g_wait(i0, rows0, sg0)
            scat(i0, rows0, sc0)
            scat_drain(i0, rows0, sc0)
            g_issue(i0 + 2, rows0, sg0)
            g_wait(i0 + 1, rows1, sg1)
            scat(i0 + 1, rows1, sc1)
            scat_drain(i0 + 1, rows1, sc1)
            return 0
        lax.fori_loop(0, (NB - 1) // 2, body, 0)
        g_wait(NB - 1, rows0, sg0)
        scat(NB - 1, rows0, sc0)
        scat_drain(NB - 1, rows0, sc0)
        plsc.subcore_barrier()

        pltpu.sync_copy(acc.at[pl.ds(s * RPT, RPT)], out_hbm.at[c, s])

    return k(h, src3, dst3).reshape(NC, N, D)


def _sc_dots(zs, zt, src3, dst3):
    @functools.partial(
        pl.kernel,
        out_type=(jax.ShapeDtypeStruct((NW, 1, EPW), jnp.float32),
                  jax.ShapeDtypeStruct((NC, NS, N), jnp.float32)),
        mesh=_mesh,
        compiler_params=_sc_params,
        scratch_types=[
            pltpu.VMEM((NB, EB), jnp.int32),
            pltpu.VMEM((NB, EB), jnp.int32),
            pltpu.VMEM((EB, D), jnp.float32),
            pltpu.VMEM((EB, D), jnp.float32),
            pltpu.VMEM((EB, D), jnp.float32),
            pltpu.VMEM((EB, D), jnp.float32),
            pltpu.VMEM((1, EPW), jnp.float32),
            pltpu.VMEM((N,), jnp.float32),
            pltpu.VMEM((L * L,), jnp.float32),
            pltpu.SemaphoreType.DMA,
            pltpu.SemaphoreType.DMA,
            pltpu.SemaphoreType.DMA,
            pltpu.SemaphoreType.DMA,
        ],
    )
    def k(zs_hbm, zt_hbm, src_hbm, dst_hbm, e_hbm, mxp_hbm,
          sidx, didx, sr0, sr1, tr0, tr1, ebuf, mxacc, ps,
          ss0, ss1, st0, st1):
        c = lax.axis_index("c")
        s = lax.axis_index("s")
        w = _wid(c, s)
        iota = jnp.arange(L, dtype=jnp.int32)

        pltpu.sync_copy(src_hbm.at[w], sidx)
        pltpu.sync_copy(dst_hbm.at[w], didx)

        def zmx(i, _):
            mxacc[pl.ds(i * L, L)] = jnp.full((L,), -jnp.inf, jnp.float32)
            return 0
        lax.fori_loop(0, N // L, zmx, 0)

        def g_issue(i, srows, trows, sems, semt):
            pltpu.async_copy(zs_hbm.at[sidx.at[i]], srows, sems)
            pltpu.async_copy(zt_hbm.at[didx.at[i]], trows, semt)

        def g_wait(i, srows, trows, sems, semt):
            pltpu.make_async_copy(zs_hbm.at[sidx.at[i]], srows, sems).wait()
            pltpu.make_async_copy(zt_hbm.at[didx.at[i]], trows, semt).wait()

        def compute(i, srows, trows):
            for g in range(GPB):
                for j in range(L):
                    acc = jnp.zeros((L,), jnp.float32)
                    for fc in range(D // L):
                        vs = srows[g * L + j, pl.ds(fc * L, L)]
                        vt = trows[g * L + j, pl.ds(fc * L, L)]
                        acc = acc + vs * vt
                    ps[pl.ds(j * L, L)] = acc
                ev = jnp.zeros((L,), jnp.float32)
                for l in range(L):
                    ev = ev + plsc.load_gather(ps, [iota * L + l])
                ebuf[0, pl.ds(i * EB + g * L, L)] = ev

                sg = sidx[i, pl.ds(g * L, L)]
                m = ev
                for r in range(1, L):
                    perm = (iota + r) & (L - 1)
                    sr = sg.at[perm].get(mode="promise_in_bounds")
                    er = ev.at[perm].get(mode="promise_in_bounds")
                    m = jnp.where(sr == sg, jnp.maximum(m, er), m)
                cur = plsc.load_gather(mxacc, [sg])
                plsc.store_scatter(mxacc, [sg], jnp.maximum(cur, m))

        g_issue(0, sr0, tr0, ss0, st0)

        def body(kk, _):
            i0 = 2 * kk
            g_issue(i0 + 1, sr1, tr1, ss1, st1)
            g_wait(i0, sr0, tr0, ss0, st0)
            compute(i0, sr0, tr0)
            g_issue(i0 + 2, sr0, tr0, ss0, st0)
            g_wait(i0 + 1, sr1, tr1, ss1, st1)
            compute(i0 + 1, sr1, tr1)
            return 0
        lax.fori_loop(0, (NB - 1) // 2, body, 0)
        g_wait(NB - 1, sr0, tr0, ss0, st0)
        compute(NB - 1, sr0, tr0)

        pltpu.sync_copy(ebuf, e_hbm.at[w])
        pltpu.sync_copy(mxacc, mxp_hbm.at[c, s])

    return k(zs, zt, src3, dst3)


def _sc_expsum(e3, src3, mx):
    @functools.partial(
        pl.kernel,
        out_type=(jax.ShapeDtypeStruct((NW, 1, EPW), jnp.float32),
                  jax.ShapeDtypeStruct((NC, NS, N), jnp.float32)),
        mesh=_mesh,
        compiler_params=_sc_params,
        scratch_types=[
            pltpu.VMEM((1, EPW), jnp.int32),
            pltpu.VMEM((1, EPW), jnp.float32),
            pltpu.VMEM((1, EPW), jnp.float32),
            pltpu.VMEM((N,), jnp.float32),
            pltpu.VMEM((N,), jnp.float32),
        ],
    )
    def k(e_hbm, src_hbm, mx_hbm, esh_hbm, dp_hbm,
          sidx, ebuf, oebuf, mxl, dacc):
        c = lax.axis_index("c")
        s = lax.axis_index("s")
        w = _wid(c, s)

        pltpu.sync_copy(src_hbm.at[w], sidx)
        pltpu.sync_copy(e_hbm.at[w], ebuf)
        pltpu.sync_copy(mx_hbm, mxl)

        def zd(i, _):
            dacc[pl.ds(i * L, L)] = jnp.zeros((L,), jnp.float32)
            return 0
        lax.fori_loop(0, N // L, zd, 0)

        def body(g, _):
            sg = sidx[0, pl.ds(g * L, L)]
            ev = ebuf[0, pl.ds(g * L, L)]
            mval = plsc.load_gather(mxl, [sg])
            esh = ev - mval
            oebuf[0, pl.ds(g * L, L)] = esh
            plsc.addupdate_scatter(dacc, [sg], jnp.exp(esh))
            return 0
        lax.fori_loop(0, EPW // L, body, 0)

        pltpu.sync_copy(oebuf, esh_hbm.at[w])
        pltpu.sync_copy(dacc, dp_hbm.at[c, s])

    return k(e3, src3, mx)


def _sc_final(esh3, src3, ld):
    @functools.partial(
        pl.kernel,
        out_type=jax.ShapeDtypeStruct((NW, 1, EPW), jnp.float32),
        mesh=_mesh,
        compiler_params=_sc_params,
        scratch_types=[
            pltpu.VMEM((1, EPW), jnp.int32),
            pltpu.VMEM((1, EPW), jnp.float32),
            pltpu.VMEM((1, EPW), jnp.float32),
            pltpu.VMEM((N,), jnp.float32),
        ],
    )
    def k(esh_hbm, src_hbm, ld_hbm, out_hbm, sidx, ebuf, obuf, ldl):
        c = lax.axis_index("c")
        s = lax.axis_index("s")
        w = _wid(c, s)

        pltpu.sync_copy(src_hbm.at[w], sidx)
        pltpu.sync_copy(esh_hbm.at[w], ebuf)
        pltpu.sync_copy(ld_hbm, ldl)

        def body(g, _):
            sg = sidx[0, pl.ds(g * L, L)]
            ev = ebuf[0, pl.ds(g * L, L)]
            obuf[0, pl.ds(g * L, L)] = ev - plsc.load_gather(ldl, [sg])
            return 0
        lax.fori_loop(0, EPW // L, body, 0)

        pltpu.sync_copy(obuf, out_hbm.at[w])

    return k(esh3, src3, ld)


_RB = 400


def _enc(x, W, b):
    def k(x_ref, w_ref, b_ref, o_ref):
        o_ref[...] = jnp.dot(x_ref[...], w_ref[...],
                             preferred_element_type=jnp.float32) + b_ref[...]

    return pl.pallas_call(
        k,
        grid=(N // _RB,),
        in_specs=[
            pl.BlockSpec((_RB, D), lambda i: (i, 0)),
            pl.BlockSpec((D, D), lambda i: (0, 0)),
            pl.BlockSpec((1, D), lambda i: (0, 0)),
        ],
        out_specs=pl.BlockSpec((_RB, D), lambda i: (i, 0)),
        out_shape=jax.ShapeDtypeStruct((N, D), jnp.float32),
    )(x, W, b.reshape(1, D))


def _proc(h, aggp, W1, b1, W2, b2, Ws, bs, Wt, bt):
    def k(h_ref, a_ref, w1, b1r, w2, b2r, ws, bsr, wt, btr, zs_ref, zt_ref):
        z = h_ref[...] + a_ref[0] + a_ref[1]
        z = jnp.dot(z, w1[...], preferred_element_type=jnp.float32) + b1r[...]
        z = jnp.maximum(z, 0.0)
        h2 = jnp.dot(z, w2[...], preferred_element_type=jnp.float32) + b2r[...]
        zs_ref[...] = jnp.dot(h2, ws[...],
                              preferred_element_type=jnp.float32) + bsr[...]
        zt_ref[...] = jnp.dot(h2, wt[...],
                              preferred_element_type=jnp.float32) + btr[...]

    wspec = pl.BlockSpec((D, D), lambda i: (0, 0))
    bspec = pl.BlockSpec((1, D), lambda i: (0, 0))
    return pl.pallas_call(
        k,
        grid=(N // _RB,),
        in_specs=[
            pl.BlockSpec((_RB, D), lambda i: (i, 0)),
            pl.BlockSpec((NC, _RB, D), lambda i: (0, i, 0)),
            wspec, bspec, wspec, bspec, wspec, bspec, wspec, bspec,
        ],
        out_specs=[pl.BlockSpec((_RB, D), lambda i: (i, 0)),
                   pl.BlockSpec((_RB, D), lambda i: (i, 0))],
        out_shape=[jax.ShapeDtypeStruct((N, D), jnp.float32),
                   jax.ShapeDtypeStruct((N, D), jnp.float32)],
    )(h, aggp, W1, b1.reshape(1, D), W2, b2.reshape(1, D),
      Ws, bs.reshape(1, D), Wt, bt.reshape(1, D))


def _combine_max(mxp):
    def k(p_ref, o_ref):
        m = jnp.max(p_ref[...], axis=(0, 1))
        o_ref[...] = jnp.where(jnp.isfinite(m), m, 0.0).reshape(1, N)

    return pl.pallas_call(
        k,
        out_shape=jax.ShapeDtypeStruct((1, N), jnp.float32),
    )(mxp)


def _combine_logsum(dp):
    def k(p_ref, o_ref):
        d = jnp.sum(p_ref[...], axis=(0, 1))
        o_ref[...] = jnp.log(d + 1e-12).reshape(1, N)

    return pl.pallas_call(
        k,
        out_shape=jax.ShapeDtypeStruct((1, N), jnp.float32),
    )(dp)


def kernel(x, edge_index, W_enc, b_enc, W1, b1, W2, b2, Ws, bs, Wt, bt):
    srcb = edge_index[0].reshape(NW, NB, EB)
    srcf = edge_index[0].reshape(NW, 1, EPW)
    dstb = edge_index[1].reshape(NW, NB, EB)
    dstf = edge_index[1].reshape(NW, 1, EPW)
    h = _enc(x, W_enc, b_enc)
    aggp = _sc_segsum(h, srcb, dstf)
    zs, zt = _proc(h, aggp, W1, b1, W2, b2, Ws, bs, Wt, bt)
    e3, mxp = _sc_dots(zs, zt, srcb, dstb)
    mx = _combine_max(mxp).reshape(N)
    esh3, dp = _sc_expsum(e3, srcf, mx)
    ld = _combine_logsum(dp).reshape(N)
    return _sc_final(esh3, srcf, ld).reshape(E)


# speedup vs baseline: 20.0406x; 1.6606x over previous
import functools

import jax
import jax.numpy as jnp
from jax import lax
from jax.experimental import pallas as pl
from jax.experimental.pallas import tpu as pltpu
from jax.experimental.pallas import tpu_sc as plsc

N = 10000
E = 320000
D = 128

NC = 2
NS = 16
NW = NC * NS
EPW = E // NW
EB = 80
NB = EPW // EB
RPT = N // NS
L = 16
GPB = EB // L
NP = 10240

_mesh = plsc.VectorSubcoreMesh(core_axis_name="c", subcore_axis_name="s")
_sc_params = pltpu.CompilerParams(needs_layout_passes=False)


def _wid(c, s):
    return c * NS + s


def _sc_segsum(h, src3, dst3):
    @functools.partial(
        pl.kernel,
        out_type=jax.ShapeDtypeStruct((NC, NS, RPT, D), jnp.float32),
        mesh=_mesh,
        compiler_params=_sc_params,
        scratch_types=[
            pltpu.VMEM((NB, EB), jnp.int32),
            pltpu.VMEM((1, EPW), jnp.int32),
            pltpu.VMEM((EB, D), jnp.float32),
            pltpu.VMEM((EB, D), jnp.float32),
            pltpu.VMEM((5, D), jnp.float32),
            pltpu.VMEM_SHARED((N, D), jnp.float32),
            pltpu.SemaphoreType.DMA,
            pltpu.SemaphoreType.DMA,
            pltpu.SemaphoreType.DMA,
            pltpu.SemaphoreType.DMA,
        ],
    )
    def k(h_hbm, src_hbm, dst_hbm, out_hbm,
          sidx, didx, rows0, rows1, zbuf, acc, sg0, sg1, sc0, sc1):
        c = lax.axis_index("c")
        s = lax.axis_index("s")
        w = _wid(c, s)

        pltpu.sync_copy(src_hbm.at[w], sidx)
        pltpu.sync_copy(dst_hbm.at[w], didx)

        def zb(i, _):
            def zb2(j, _):
                zbuf[i, pl.ds(j * L, L)] = jnp.zeros((L,), jnp.float32)
                return 0
            return lax.fori_loop(0, D // L, zb2, 0)
        lax.fori_loop(0, 5, zb, 0)

        def zacc(i, _):
            pltpu.sync_copy(zbuf, acc.at[pl.ds(s * RPT + i * 5, 5)])
            return 0
        lax.fori_loop(0, RPT // 5, zacc, 0)
        plsc.subcore_barrier()

        def g_issue(i, buf, sem):
            pltpu.async_copy(h_hbm.at[sidx.at[i]], buf, sem)

        def g_wait(i, buf, sem):
            pltpu.make_async_copy(h_hbm.at[sidx.at[i]], buf, sem).wait()

        def scat(i, buf, sem):
            for g in range(GPB):
                dg = didx[0, pl.ds(i * EB + g * L, L)]
                pltpu.async_copy(buf.at[pl.ds(g * L, L)], acc.at[dg], sem,
                                 add=True)

        def scat_drain(i, buf, sem):
            for g in range(GPB):
                dg = didx[0, pl.ds(i * EB + g * L, L)]
                pltpu.make_async_copy(buf.at[pl.ds(g * L, L)], acc.at[dg],
                                      sem).wait()

        g_issue(0, rows0, sg0)

        def body(kk, _):
            i0 = 2 * kk
            g_issue(i0 + 1, rows1, sg1)
            g_wait(i0, rows0, sg0)
            scat(i0, rows0, sc0)
            scat_drain(i0, rows0, sc0)
            g_issue(i0 + 2, rows0, sg0)
            g_wait(i0 + 1, rows1, sg1)
            scat(i0 + 1, rows1, sc1)
            scat_drain(i0 + 1, rows1, sc1)
            return 0
        lax.fori_loop(0, (NB - 1) // 2, body, 0)
        g_wait(NB - 1, rows0, sg0)
        scat(NB - 1, rows0, sc0)
        scat_drain(NB - 1, rows0, sc0)
        plsc.subcore_barrier()

        pltpu.sync_copy(acc.at[pl.ds(s * RPT, RPT)], out_hbm.at[c, s])

    return k(h, src3, dst3).reshape(NC, N, D)


def _sc_dots(zs, zt, src3, dst3):
    @functools.partial(
        pl.kernel,
        out_type=(jax.ShapeDtypeStruct((NW, 1, EPW), jnp.float32),
                  jax.ShapeDtypeStruct((NC, NS, NP), jnp.float32)),
        mesh=_mesh,
        compiler_params=_sc_params,
        scratch_types=[
            pltpu.VMEM((NB, EB), jnp.int32),
            pltpu.VMEM((NB, EB), jnp.int32),
            pltpu.VMEM((EB, D), jnp.float32),
            pltpu.VMEM((EB, D), jnp.float32),
            pltpu.VMEM((EB, D), jnp.float32),
            pltpu.VMEM((EB, D), jnp.float32),
            pltpu.VMEM((1, EPW), jnp.float32),
            pltpu.VMEM((NP,), jnp.float32),
            pltpu.VMEM((L * L,), jnp.float32),
            pltpu.SemaphoreType.DMA,
            pltpu.SemaphoreType.DMA,
            pltpu.SemaphoreType.DMA,
            pltpu.SemaphoreType.DMA,
        ],
    )
    def k(zs_hbm, zt_hbm, src_hbm, dst_hbm, e_hbm, mxp_hbm,
          sidx, didx, sr0, sr1, tr0, tr1, ebuf, mxacc, ps,
          ss0, ss1, st0, st1):
        c = lax.axis_index("c")
        s = lax.axis_index("s")
        w = _wid(c, s)
        iota = jnp.arange(L, dtype=jnp.int32)

        pltpu.sync_copy(src_hbm.at[w], sidx)
        pltpu.sync_copy(dst_hbm.at[w], didx)

        def zmx(i, _):
            mxacc[pl.ds(i * L, L)] = jnp.full((L,), -jnp.inf, jnp.float32)
            return 0
        lax.fori_loop(0, NP // L, zmx, 0)

        def g_issue(i, srows, trows, sems, semt):
            pltpu.async_copy(zs_hbm.at[sidx.at[i]], srows, sems)
            pltpu.async_copy(zt_hbm.at[didx.at[i]], trows, semt)

        def g_wait(i, srows, trows, sems, semt):
            pltpu.make_async_copy(zs_hbm.at[sidx.at[i]], srows, sems).wait()
            pltpu.make_async_copy(zt_hbm.at[didx.at[i]], trows, semt).wait()

        def compute(i, srows, trows):
            def grp(g, _):
                def fcb(fc, accs):
                    return tuple(
                        accs[j] + srows[g * L + j, pl.ds(fc * L, L)]
                        * trows[g * L + j, pl.ds(fc * L, L)]
                        for j in range(L))
                accs = lax.fori_loop(
                    0, D // L, fcb,
                    tuple(jnp.zeros((L,), jnp.float32) for _ in range(L)))
                for j in range(L):
                    ps[pl.ds(j * L, L)] = accs[j]
                ev = jnp.zeros((L,), jnp.float32)
                for l in range(L):
                    ev = ev + plsc.load_gather(ps, [iota * L + l])
                ebuf[0, pl.ds(i * EB + g * L, L)] = ev

                sg = sidx[i, pl.ds(g * L, L)]
                m = ev
                for r in range(1, L):
                    perm = (iota + r) & (L - 1)
                    sr = sg.at[perm].get(mode="promise_in_bounds")
                    er = ev.at[perm].get(mode="promise_in_bounds")
                    m = jnp.where(sr == sg, jnp.maximum(m, er), m)
                cur = plsc.load_gather(mxacc, [sg])
                plsc.store_scatter(mxacc, [sg], jnp.maximum(cur, m))
                return 0
            lax.fori_loop(0, GPB, grp, 0)

        g_issue(0, sr0, tr0, ss0, st0)

        def body(kk, _):
            i0 = 2 * kk
            g_issue(i0 + 1, sr1, tr1, ss1, st1)
            g_wait(i0, sr0, tr0, ss0, st0)
            compute(i0, sr0, tr0)
            g_issue(i0 + 2, sr0, tr0, ss0, st0)
            g_wait(i0 + 1, sr1, tr1, ss1, st1)
            compute(i0 + 1, sr1, tr1)
            return 0
        lax.fori_loop(0, (NB - 1) // 2, body, 0)
        g_wait(NB - 1, sr0, tr0, ss0, st0)
        compute(NB - 1, sr0, tr0)

        pltpu.sync_copy(ebuf, e_hbm.at[w])
        pltpu.sync_copy(mxacc, mxp_hbm.at[c, s])

    return k(zs, zt, src3, dst3)


CW = NP // NS


def _sc_expsum(e3, src3, mxp32):
    @functools.partial(
        pl.kernel,
        out_type=(jax.ShapeDtypeStruct((NW, 1, EPW), jnp.float32),
                  jax.ShapeDtypeStruct((NC, NS, NP), jnp.float32)),
        mesh=_mesh,
        compiler_params=_sc_params,
        scratch_types=[
            pltpu.VMEM((1, EPW), jnp.int32),
            pltpu.VMEM((1, EPW), jnp.float32),
            pltpu.VMEM((1, EPW), jnp.float32),
            pltpu.VMEM((NP,), jnp.float32),
            pltpu.VMEM((NP,), jnp.float32),
            pltpu.VMEM((NW, CW), jnp.float32),
            pltpu.VMEM((CW,), jnp.float32),
            pltpu.VMEM_SHARED((NP,), jnp.float32),
        ],
    )
    def k(e_hbm, src_hbm, mxp_hbm, esh_hbm, dp_hbm,
          sidx, ebuf, oebuf, mxl, dacc, pbuf, combuf, sh):
        c = lax.axis_index("c")
        s = lax.axis_index("s")
        w = _wid(c, s)

        pltpu.sync_copy(src_hbm.at[w], sidx)
        pltpu.sync_copy(e_hbm.at[w], ebuf)

        pltpu.sync_copy(mxp_hbm.at[:, pl.ds(s * CW, CW)], pbuf)

        def cmx(kk, _):
            m = pbuf[0, pl.ds(kk * L, L)]
            for r in range(1, NW):
                m = jnp.maximum(m, pbuf[r, pl.ds(kk * L, L)])
            neg = jnp.full((L,), -jnp.inf, jnp.float32)
            combuf[pl.ds(kk * L, L)] = jnp.where(
                m > neg, m, jnp.zeros((L,), jnp.float32))
            return 0
        lax.fori_loop(0, CW // L, cmx, 0)
        pltpu.sync_copy(combuf, sh.at[pl.ds(s * CW, CW)])
        plsc.subcore_barrier()
        pltpu.sync_copy(sh, mxl)

        def zd(i, _):
            dacc[pl.ds(i * L, L)] = jnp.zeros((L,), jnp.float32)
            return 0
        lax.fori_loop(0, NP // L, zd, 0)

        def body(g, _):
            sg = sidx[0, pl.ds(g * L, L)]
            ev = ebuf[0, pl.ds(g * L, L)]
            mval = plsc.load_gather(mxl, [sg])
            esh = ev - mval
            oebuf[0, pl.ds(g * L, L)] = esh
            plsc.addupdate_scatter(dacc, [sg], jnp.exp(esh))
            return 0
        lax.fori_loop(0, EPW // L, body, 0)

        pltpu.sync_copy(oebuf, esh_hbm.at[w])
        pltpu.sync_copy(dacc, dp_hbm.at[c, s])

    return k(e3, src3, mxp32)


def _sc_final(esh3, src3, dp32):
    @functools.partial(
        pl.kernel,
        out_type=jax.ShapeDtypeStruct((NW, 1, EPW), jnp.float32),
        mesh=_mesh,
        compiler_params=_sc_params,
        scratch_types=[
            pltpu.VMEM((1, EPW), jnp.int32),
            pltpu.VMEM((1, EPW), jnp.float32),
            pltpu.VMEM((1, EPW), jnp.float32),
            pltpu.VMEM((NP,), jnp.float32),
            pltpu.VMEM((NW, CW), jnp.float32),
            pltpu.VMEM((CW,), jnp.float32),
            pltpu.VMEM_SHARED((NP,), jnp.float32),
        ],
    )
    def k(esh_hbm, src_hbm, dp_hbm, out_hbm,
          sidx, ebuf, obuf, ldl, pbuf, combuf, sh):
        c = lax.axis_index("c")
        s = lax.axis_index("s")
        w = _wid(c, s)

        pltpu.sync_copy(src_hbm.at[w], sidx)
        pltpu.sync_copy(esh_hbm.at[w], ebuf)

        pltpu.sync_copy(dp_hbm.at[:, pl.ds(s * CW, CW)], pbuf)

        def cld(kk, _):
            d = pbuf[0, pl.ds(kk * L, L)]
            for r in range(1, NW):
                d = d + pbuf[r, pl.ds(kk * L, L)]
            d = d + jnp.full((L,), 1e-12, jnp.float32)
            bits = plsc.bitcast(d, jnp.int32)
            ex = (bits >> 23) - 127
            mant = plsc.bitcast(
                (bits & 0x007FFFFF) | 0x3F800000, jnp.float32)
            one_v = jnp.full((L,), 1.0, jnp.float32)
            z = (mant - one_v) / (mant + one_v)
            z2 = z * z
            p = 1.0 / 7.0 + z2 * (1.0 / 9.0)
            p = 1.0 / 5.0 + z2 * p
            p = 1.0 / 3.0 + z2 * p
            lnm = 2.0 * z * (one_v + z2 * p)
            ln2 = jnp.full((L,), 0.6931471805599453, jnp.float32)
            combuf[pl.ds(kk * L, L)] = ex.astype(jnp.float32) * ln2 + lnm
            return 0
        lax.fori_loop(0, CW // L, cld, 0)
        pltpu.sync_copy(combuf, sh.at[pl.ds(s * CW, CW)])
        plsc.subcore_barrier()
        pltpu.sync_copy(sh, ldl)

        def body(g, _):
            sg = sidx[0, pl.ds(g * L, L)]
            ev = ebuf[0, pl.ds(g * L, L)]
            obuf[0, pl.ds(g * L, L)] = ev - plsc.load_gather(ldl, [sg])
            return 0
        lax.fori_loop(0, EPW // L, body, 0)

        pltpu.sync_copy(obuf, out_hbm.at[w])

    return k(esh3, src3, dp32)


_RB = 400


def _enc(x, W, b):
    def k(x_ref, w_ref, b_ref, o_ref):
        o_ref[...] = jnp.dot(x_ref[...], w_ref[...],
                             preferred_element_type=jnp.float32) + b_ref[...]

    return pl.pallas_call(
        k,
        grid=(N // _RB,),
        in_specs=[
            pl.BlockSpec((_RB, D), lambda i: (i, 0)),
            pl.BlockSpec((D, D), lambda i: (0, 0)),
            pl.BlockSpec((1, D), lambda i: (0, 0)),
        ],
        out_specs=pl.BlockSpec((_RB, D), lambda i: (i, 0)),
        out_shape=jax.ShapeDtypeStruct((N, D), jnp.float32),
    )(x, W, b.reshape(1, D))


def _proc(h, aggp, W1, b1, W2, b2, Ws, bs, Wt, bt):
    def k(h_ref, a_ref, w1, b1r, w2, b2r, ws, bsr, wt, btr, zs_ref, zt_ref):
        z = h_ref[...] + a_ref[0] + a_ref[1]
        z = jnp.dot(z, w1[...], preferred_element_type=jnp.float32) + b1r[...]
        z = jnp.maximum(z, 0.0)
        h2 = jnp.dot(z, w2[...], preferred_element_type=jnp.float32) + b2r[...]
        zs_ref[...] = jnp.dot(h2, ws[...],
                              preferred_element_type=jnp.float32) + bsr[...]
        zt_ref[...] = jnp.dot(h2, wt[...],
                              preferred_element_type=jnp.float32) + btr[...]

    wspec = pl.BlockSpec((D, D), lambda i: (0, 0))
    bspec = pl.BlockSpec((1, D), lambda i: (0, 0))
    return pl.pallas_call(
        k,
        grid=(N // _RB,),
        in_specs=[
            pl.BlockSpec((_RB, D), lambda i: (i, 0)),
            pl.BlockSpec((NC, _RB, D), lambda i: (0, i, 0)),
            wspec, bspec, wspec, bspec, wspec, bspec, wspec, bspec,
        ],
        out_specs=[pl.BlockSpec((_RB, D), lambda i: (i, 0)),
                   pl.BlockSpec((_RB, D), lambda i: (i, 0))],
        out_shape=[jax.ShapeDtypeStruct((N, D), jnp.float32),
                   jax.ShapeDtypeStruct((N, D), jnp.float32)],
    )(h, aggp, W1, b1.reshape(1, D), W2, b2.reshape(1, D),
      Ws, bs.reshape(1, D), Wt, bt.reshape(1, D))


def kernel(x, edge_index, W_enc, b_enc, W1, b1, W2, b2, Ws, bs, Wt, bt):
    srcb = edge_index[0].reshape(NW, NB, EB)
    srcf = edge_index[0].reshape(NW, 1, EPW)
    dstb = edge_index[1].reshape(NW, NB, EB)
    dstf = edge_index[1].reshape(NW, 1, EPW)
    h = _enc(x, W_enc, b_enc)
    aggp = _sc_segsum(h, srcb, dstf)
    zs, zt = _proc(h, aggp, W1, b1, W2, b2, Ws, bs, Wt, bt)
    e3, mxp = _sc_dots(zs, zt, srcb, dstb)
    esh3, dp = _sc_expsum(e3, srcf, mxp.reshape(NW, NP))
    return _sc_final(esh3, srcf, dp.reshape(NW, NP)).reshape(E)
